# baseline (speedup 1.0000x reference)
import functools

import jax
import jax.numpy as jnp
from jax import lax
from jax.experimental import pallas as pl
from jax.experimental.pallas import tpu as pltpu
from jax.experimental.pallas import tpu_sc as plsc

NB = 50000
NU = 400000
NUM_BLOCK_TYPE = 32
NUM_ATOM_TYPE = 128
EMBED = 128

NC = 2
NS = 16
NW = NC * NS
L = 16

CHUNK = 128
NCHUNK = NU // CHUNK
BASE_CHUNKS = NCHUNK // NW
EXTRA = NCHUNK % NW


def _build_fused(block_table, atom_table):
    def body(b_ref, a_ref, o_ref):
        t = pl.program_id(0)
        o_ref[...] = a_ref[...] + b_ref[pl.ds(t, 1), :]

    return pl.pallas_call(
        body,
        grid=(NUM_BLOCK_TYPE,),
        in_specs=[
            pl.BlockSpec((NUM_BLOCK_TYPE, EMBED), lambda i: (0, 0)),
            pl.BlockSpec((NUM_ATOM_TYPE, EMBED), lambda i: (0, 0)),
        ],
        out_specs=pl.BlockSpec((NUM_ATOM_TYPE, EMBED), lambda i: (i, 0)),
        out_shape=jax.ShapeDtypeStruct(
            (NUM_BLOCK_TYPE * NUM_ATOM_TYPE, EMBED), jnp.float32
        ),
    )(block_table, atom_table)


K = 4
SSTEP = NW * K
NSUPER = (NCHUNK + SSTEP - 1) // SSTEP


def _make_sc_gather():
    mesh = plsc.VectorSubcoreMesh(core_axis_name="c", subcore_axis_name="s")
    NSUPER2 = NSUPER + (NSUPER % 2)

    @functools.partial(
        pl.kernel,
        mesh=mesh,
        out_type=jax.ShapeDtypeStruct((NU, EMBED), jnp.float32),
        scratch_types=[
            pltpu.VMEM((K, CHUNK), jnp.int32),
            pltpu.VMEM((K, CHUNK), jnp.int32),
            pltpu.VMEM((K, CHUNK), jnp.int32),
            pltpu.VMEM((K, CHUNK), jnp.int32),
            pltpu.VMEM((K, CHUNK), jnp.int32),
            pltpu.VMEM((K, CHUNK), jnp.int32),
            pltpu.VMEM((K * CHUNK, EMBED), jnp.float32),
            pltpu.VMEM_SHARED(
                (NUM_BLOCK_TYPE * NUM_ATOM_TYPE, EMBED), jnp.float32
            ),
            pltpu.VMEM_SHARED((NB,), jnp.int32),
            pltpu.SemaphoreType.DMA,
            pltpu.SemaphoreType.DMA,
            pltpu.SemaphoreType.DMA,
            (pltpu.SemaphoreType.DMA,) * K,
            (pltpu.SemaphoreType.DMA,) * K,
        ],
    )
    def sc_gather(
        s_hbm, bid_hbm, a_hbm, fused_hbm, out_hbm,
        bid0, bid1, a0, a1, t_v, f_v, rows_v, fused_sh, s_sh,
        sin0, sin1, sem_t, sem_g, sem_o,
    ):
        bid_vs = (bid0, bid1)
        a_vs = (a0, a1)
        sem_in = (sin0, sin1)

        cid = lax.axis_index("c")
        sid = lax.axis_index("s")
        wid = sid * NC + cid

        def chunk_of(s, k):
            return s * SSTEP + wid * K + k

        def active(s, k):
            return jnp.logical_and(s >= 0, chunk_of(s, k) < NCHUNK)

        def row0_of(s, k):
            return chunk_of(s, k) * CHUNK

        def fire_in(s, b):
            for k in range(K):
                @pl.when(active(s, k))
                def _():
                    row0 = row0_of(s, k)
                    pltpu.async_copy(
                        bid_hbm.at[pl.ds(row0, CHUNK)], bid_vs[b].at[k], sem_in[b]
                    )
                    pltpu.async_copy(
                        a_hbm.at[pl.ds(row0, CHUNK)], a_vs[b].at[k], sem_in[b]
                    )

        def do_superstep(s, b):
            for k in range(K):
                @pl.when(active(s, k))
                def _():
                    row0 = row0_of(s, k)
                    pltpu.make_async_copy(
                        bid_hbm.at[pl.ds(row0, CHUNK)], bid_vs[b].at[k], sem_in[b]
                    ).wait()
                    pltpu.make_async_copy(
                        a_hbm.at[pl.ds(row0, CHUNK)], a_vs[b].at[k], sem_in[b]
                    ).wait()
                    pltpu.async_copy(s_sh.at[bid_vs[b].at[k]], t_v.at[k], sem_t)

            for k in range(K):
                @pl.when(active(s, k))
                def _():
                    pltpu.make_async_copy(
                        s_sh.at[bid_vs[b].at[k]], t_v.at[k], sem_t
                    ).wait()
                    for g in range(CHUNK // L):
                        t16 = t_v[k, pl.ds(g * L, L)]
                        a16 = a_vs[b][k, pl.ds(g * L, L)]
                        f_v[k, pl.ds(g * L, L)] = t16 * EMBED + a16

            fire_in(s + 2, b)

            for k in range(K):
                @pl.when(active(s - 1, k))
                def _():
                    pltpu.make_async_copy(
                        rows_v.at[pl.ds(k * CHUNK, CHUNK)],
                        out_hbm.at[pl.ds(row0_of(s - 1, k), CHUNK)],
                        sem_o[k],
                    ).wait()

                @pl.when(active(s, k))
                def _():
                    fsrc = fused_sh if k % 2 == 0 else fused_hbm
                    pltpu.async_copy(
                        fsrc.at[f_v.at[k]],
                        rows_v.at[pl.ds(k * CHUNK, CHUNK)],
                        sem_g[k],
                    )

            for k in range(K):
                @pl.when(active(s, k))
                def _():
                    fsrc = fused_sh if k % 2 == 0 else fused_hbm
                    pltpu.make_async_copy(
                        fsrc.at[f_v.at[k]],
                        rows_v.at[pl.ds(k * CHUNK, CHUNK)],
                        sem_g[k],
                    ).wait()
                    pltpu.async_copy(
                        rows_v.at[pl.ds(k * CHUNK, CHUNK)],
                        out_hbm.at[pl.ds(row0_of(s, k), CHUNK)],
                        sem_o[k],
                    )

        @pl.when(sid == 0)
        def _stage():
            pltpu.sync_copy(fused_hbm, fused_sh)
            pltpu.sync_copy(s_hbm, s_sh)

        plsc.subcore_barrier()

        fire_in(0, 0)
        fire_in(1, 1)

        @pl.loop(0, NSUPER2, step=2)
        def _body(s0):
            for db in range(2):
                do_superstep(s0 + db, db)

        for k in range(K):
            @pl.when(active(NSUPER2 - 1, k))
            def _drain():
                pltpu.make_async_copy(
                    rows_v.at[pl.ds(k * CHUNK, CHUNK)],
                    out_hbm.at[pl.ds(row0_of(NSUPER2 - 1, k), CHUNK)],
                    sem_o[k],
                ).wait()

    return sc_gather


_sc_gather = _make_sc_gather()


@jax.jit
def kernel(S, A, block_id, block_table, atom_table):
    fused = _build_fused(block_table, atom_table)
    return _sc_gather(S, block_id, A, fused)


# speedup vs baseline: 1.2605x; 1.2605x over previous
import functools

import jax
import jax.numpy as jnp
from jax import lax
from jax.experimental import pallas as pl
from jax.experimental.pallas import tpu as pltpu
from jax.experimental.pallas import tpu_sc as plsc

NB = 50000
NU = 400000
NUM_BLOCK_TYPE = 32
NUM_ATOM_TYPE = 128
EMBED = 128

NC = 2
NS = 16
NW = NC * NS
L = 16

CHUNK = 128
NCHUNK = NU // CHUNK
BASE_CHUNKS = NCHUNK // NW
EXTRA = NCHUNK % NW


def _build_fused(block_table, atom_table):
    def body(b_ref, a_ref, o_ref):
        t = pl.program_id(0)
        o_ref[...] = a_ref[...] + b_ref[pl.ds(t, 1), :]

    return pl.pallas_call(
        body,
        grid=(NUM_BLOCK_TYPE,),
        in_specs=[
            pl.BlockSpec((NUM_BLOCK_TYPE, EMBED), lambda i: (0, 0)),
            pl.BlockSpec((NUM_ATOM_TYPE, EMBED), lambda i: (0, 0)),
        ],
        out_specs=pl.BlockSpec((NUM_ATOM_TYPE, EMBED), lambda i: (i, 0)),
        out_shape=jax.ShapeDtypeStruct(
            (NUM_BLOCK_TYPE * NUM_ATOM_TYPE, EMBED), jnp.float32
        ),
    )(block_table, atom_table)


K = 4
SSTEP = NW * K
NSUPER = (NCHUNK + SSTEP - 1) // SSTEP


def _make_sc_gather():
    mesh = plsc.VectorSubcoreMesh(core_axis_name="c", subcore_axis_name="s")
    NSUPER2 = NSUPER + (NSUPER % 2)

    @functools.partial(
        pl.kernel,
        mesh=mesh,
        out_type=jax.ShapeDtypeStruct((NU, EMBED), jnp.float32),
        scratch_types=[
            pltpu.VMEM((K, CHUNK), jnp.int32),
            pltpu.VMEM((K, CHUNK), jnp.int32),
            pltpu.VMEM((K, CHUNK), jnp.int32),
            pltpu.VMEM((K, CHUNK), jnp.int32),
            pltpu.VMEM((K, CHUNK), jnp.int32),
            pltpu.VMEM((K, CHUNK), jnp.int32),
            pltpu.VMEM((K * CHUNK, EMBED), jnp.float32),
            pltpu.VMEM_SHARED(
                (NUM_BLOCK_TYPE * NUM_ATOM_TYPE, EMBED), jnp.float32
            ),
            pltpu.VMEM_SHARED((NB,), jnp.int32),
            pltpu.SemaphoreType.DMA,
            pltpu.SemaphoreType.DMA,
            pltpu.SemaphoreType.DMA,
            (pltpu.SemaphoreType.DMA,) * K,
            (pltpu.SemaphoreType.DMA,) * K,
        ],
    )
    def sc_gather(
        s_hbm, bid_hbm, a_hbm, fused_hbm, out_hbm,
        bid0, bid1, a0, a1, t_v, f_v, rows_v, fused_sh, s_sh,
        sin0, sin1, sem_t, sem_g, sem_o,
    ):
        bid_vs = (bid0, bid1)
        a_vs = (a0, a1)
        sem_in = (sin0, sin1)

        cid = lax.axis_index("c")
        sid = lax.axis_index("s")
        wid = sid * NC + cid

        def chunk_of(s, k):
            return s * SSTEP + wid * K + k

        def active(s, k):
            return jnp.logical_and(s >= 0, chunk_of(s, k) < NCHUNK)

        def row0_of(s, k):
            return chunk_of(s, k) * CHUNK

        def fire_in(s, b):
            for k in range(K):
                @pl.when(active(s, k))
                def _():
                    row0 = row0_of(s, k)
                    pltpu.async_copy(
                        bid_hbm.at[pl.ds(row0, CHUNK)], bid_vs[b].at[k], sem_in[b]
                    )
                    pltpu.async_copy(
                        a_hbm.at[pl.ds(row0, CHUNK)], a_vs[b].at[k], sem_in[b]
                    )

        def do_superstep(s, b):
            for k in range(K):
                @pl.when(active(s, k))
                def _():
                    row0 = row0_of(s, k)
                    pltpu.make_async_copy(
                        bid_hbm.at[pl.ds(row0, CHUNK)], bid_vs[b].at[k], sem_in[b]
                    ).wait()
                    pltpu.make_async_copy(
                        a_hbm.at[pl.ds(row0, CHUNK)], a_vs[b].at[k], sem_in[b]
                    ).wait()
                    pltpu.async_copy(s_sh.at[bid_vs[b].at[k]], t_v.at[k], sem_t)

            for k in range(K):
                @pl.when(active(s, k))
                def _():
                    pltpu.make_async_copy(
                        s_sh.at[bid_vs[b].at[k]], t_v.at[k], sem_t
                    ).wait()
                    for g in range(CHUNK // L):
                        t16 = t_v[k, pl.ds(g * L, L)]
                        a16 = a_vs[b][k, pl.ds(g * L, L)]
                        f_v[k, pl.ds(g * L, L)] = t16 * EMBED + a16

            fire_in(s + 2, b)

            for k in range(K):
                @pl.when(active(s - 1, k))
                def _():
                    pltpu.make_async_copy(
                        rows_v.at[pl.ds(k * CHUNK, CHUNK)],
                        out_hbm.at[pl.ds(row0_of(s - 1, k), CHUNK)],
                        sem_o[k],
                    ).wait()

                @pl.when(active(s, k))
                def _():
                    pltpu.async_copy(
                        fused_sh.at[f_v.at[k]],
                        rows_v.at[pl.ds(k * CHUNK, CHUNK)],
                        sem_g[k],
                    )

            for k in range(K):
                @pl.when(active(s, k))
                def _():
                    pltpu.make_async_copy(
                        fused_sh.at[f_v.at[k]],
                        rows_v.at[pl.ds(k * CHUNK, CHUNK)],
                        sem_g[k],
                    ).wait()
                    pltpu.async_copy(
                        rows_v.at[pl.ds(k * CHUNK, CHUNK)],
                        out_hbm.at[pl.ds(row0_of(s, k), CHUNK)],
                        sem_o[k],
                    )

        @pl.when(sid == 0)
        def _stage():
            pltpu.sync_copy(fused_hbm, fused_sh)
            pltpu.sync_copy(s_hbm, s_sh)

        plsc.subcore_barrier()

        fire_in(0, 0)
        fire_in(1, 1)

        @pl.loop(0, NSUPER2, step=2)
        def _body(s0):
            for db in range(2):
                do_superstep(s0 + db, db)

        for k in range(K):
            @pl.when(active(NSUPER2 - 1, k))
            def _drain():
                pltpu.make_async_copy(
                    rows_v.at[pl.ds(k * CHUNK, CHUNK)],
                    out_hbm.at[pl.ds(row0_of(NSUPER2 - 1, k), CHUNK)],
                    sem_o[k],
                ).wait()

    return sc_gather


_sc_gather = _make_sc_gather()


@jax.jit
def kernel(S, A, block_id, block_table, atom_table):
    fused = _build_fused(block_table, atom_table)
    return _sc_gather(S, block_id, A, fused)


# speedup vs baseline: 1.3126x; 1.0414x over previous
import functools

import jax
import jax.numpy as jnp
from jax import lax
from jax.experimental import pallas as pl
from jax.experimental.pallas import tpu as pltpu
from jax.experimental.pallas import tpu_sc as plsc

NB = 50000
NU = 400000
NUM_BLOCK_TYPE = 32
NUM_ATOM_TYPE = 128
EMBED = 128

NC = 2
NS = 16
NW = NC * NS
L = 16

CHUNK = 128
NCHUNK = NU // CHUNK
BASE_CHUNKS = NCHUNK // NW
EXTRA = NCHUNK % NW


K = 4
SSTEP = NW * K
NSUPER = (NCHUNK + SSTEP - 1) // SSTEP


def _make_sc_gather():
    mesh = plsc.VectorSubcoreMesh(core_axis_name="c", subcore_axis_name="s")
    NSUPER2 = NSUPER + (NSUPER % 2)

    @functools.partial(
        pl.kernel,
        mesh=mesh,
        out_type=jax.ShapeDtypeStruct((NU, EMBED), jnp.float32),
        scratch_types=[
            pltpu.VMEM((K, CHUNK), jnp.int32),
            pltpu.VMEM((K, CHUNK), jnp.int32),
            pltpu.VMEM((K, CHUNK), jnp.int32),
            pltpu.VMEM((K, CHUNK), jnp.int32),
            pltpu.VMEM((K, CHUNK), jnp.int32),
            pltpu.VMEM((K, CHUNK), jnp.int32),
            pltpu.VMEM((K * CHUNK, EMBED), jnp.float32),
            pltpu.VMEM_SHARED(
                (NUM_BLOCK_TYPE * NUM_ATOM_TYPE, EMBED), jnp.float32
            ),
            pltpu.VMEM_SHARED((NB,), jnp.int32),
            pltpu.VMEM((NUM_ATOM_TYPE, EMBED), jnp.float32),
            pltpu.VMEM((2, EMBED), jnp.float32),
            pltpu.SemaphoreType.DMA,
            pltpu.SemaphoreType.DMA,
            pltpu.SemaphoreType.DMA,
            (pltpu.SemaphoreType.DMA,) * K,
            (pltpu.SemaphoreType.DMA,) * K,
        ],
    )
    def sc_gather(
        s_hbm, bid_hbm, a_hbm, blockt_hbm, atomt_hbm, out_hbm,
        bid0, bid1, a0, a1, t_v, f_v, rows_v, fused_sh, s_sh,
        atom_v, blk_v,
        sin0, sin1, sem_t, sem_g, sem_o,
    ):
        bid_vs = (bid0, bid1)
        a_vs = (a0, a1)
        sem_in = (sin0, sin1)

        cid = lax.axis_index("c")
        sid = lax.axis_index("s")
        wid = sid * NC + cid

        def chunk_of(s, k):
            return s * SSTEP + wid * K + k

        def active(s, k):
            return jnp.logical_and(s >= 0, chunk_of(s, k) < NCHUNK)

        def row0_of(s, k):
            return chunk_of(s, k) * CHUNK

        def fire_in(s, b):
            for k in range(K):
                @pl.when(active(s, k))
                def _():
                    row0 = row0_of(s, k)
                    pltpu.async_copy(
                        bid_hbm.at[pl.ds(row0, CHUNK)], bid_vs[b].at[k], sem_in[b]
                    )
                    pltpu.async_copy(
                        a_hbm.at[pl.ds(row0, CHUNK)], a_vs[b].at[k], sem_in[b]
                    )

        def do_superstep(s, b):
            for k in range(K):
                @pl.when(active(s, k))
                def _():
                    row0 = row0_of(s, k)
                    pltpu.make_async_copy(
                        bid_hbm.at[pl.ds(row0, CHUNK)], bid_vs[b].at[k], sem_in[b]
                    ).wait()
                    pltpu.make_async_copy(
                        a_hbm.at[pl.ds(row0, CHUNK)], a_vs[b].at[k], sem_in[b]
                    ).wait()
                    pltpu.async_copy(s_sh.at[bid_vs[b].at[k]], t_v.at[k], sem_t)

            for k in range(K):
                @pl.when(active(s, k))
                def _():
                    pltpu.make_async_copy(
                        s_sh.at[bid_vs[b].at[k]], t_v.at[k], sem_t
                    ).wait()
                    for g in range(CHUNK // L):
                        t16 = t_v[k, pl.ds(g * L, L)]
                        a16 = a_vs[b][k, pl.ds(g * L, L)]
                        f_v[k, pl.ds(g * L, L)] = t16 * EMBED + a16

            fire_in(s + 2, b)

            for k in range(K):
                @pl.when(active(s - 1, k))
                def _():
                    pltpu.make_async_copy(
                        rows_v.at[pl.ds(k * CHUNK, CHUNK)],
                        out_hbm.at[pl.ds(row0_of(s - 1, k), CHUNK)],
                        sem_o[k],
                    ).wait()

                @pl.when(active(s, k))
                def _():
                    pltpu.async_copy(
                        fused_sh.at[f_v.at[k]],
                        rows_v.at[pl.ds(k * CHUNK, CHUNK)],
                        sem_g[k],
                    )

            for k in range(K):
                @pl.when(active(s, k))
                def _():
                    pltpu.make_async_copy(
                        fused_sh.at[f_v.at[k]],
                        rows_v.at[pl.ds(k * CHUNK, CHUNK)],
                        sem_g[k],
                    ).wait()
                    pltpu.async_copy(
                        rows_v.at[pl.ds(k * CHUNK, CHUNK)],
                        out_hbm.at[pl.ds(row0_of(s, k), CHUNK)],
                        sem_o[k],
                    )

        @pl.when(sid == 0)
        def _stage_s():
            pltpu.sync_copy(s_hbm, s_sh)

        pltpu.sync_copy(atomt_hbm, atom_v)
        pltpu.sync_copy(blockt_hbm.at[pl.ds(2 * sid, 2)], blk_v)
        blk_regs = [
            [blk_v[tl, pl.ds(g * L, L)] for g in range(EMBED // L)]
            for tl in range(2)
        ]

        @pl.loop(0, NUM_ATOM_TYPE // 8)
        def _build(ab):
            for da in range(8):
                a = ab * 8 + da
                for tl in range(2):
                    for g in range(EMBED // L):
                        rows_v[tl * EMBED + a, pl.ds(g * L, L)] = (
                            atom_v[a, pl.ds(g * L, L)] + blk_regs[tl][g]
                        )

        pltpu.sync_copy(
            rows_v.at[pl.ds(0, 2 * EMBED)],
            fused_sh.at[pl.ds(sid * 2 * EMBED, 2 * EMBED)],
        )
        plsc.subcore_barrier()

        fire_in(0, 0)
        fire_in(1, 1)

        @pl.loop(0, NSUPER2, step=2)
        def _body(s0):
            for db in range(2):
                do_superstep(s0 + db, db)

        for k in range(K):
            @pl.when(active(NSUPER2 - 1, k))
            def _drain():
                pltpu.make_async_copy(
                    rows_v.at[pl.ds(k * CHUNK, CHUNK)],
                    out_hbm.at[pl.ds(row0_of(NSUPER2 - 1, k), CHUNK)],
                    sem_o[k],
                ).wait()

    return sc_gather


_sc_gather = _make_sc_gather()


@jax.jit
def kernel(S, A, block_id, block_table, atom_table):
    return _sc_gather(S, block_id, A, block_table, atom_table)


# speedup vs baseline: 1.3240x; 1.0086x over previous
import functools

import jax
import jax.numpy as jnp
from jax import lax
from jax.experimental import pallas as pl
from jax.experimental.pallas import tpu as pltpu
from jax.experimental.pallas import tpu_sc as plsc

NB = 50000
NU = 400000
NUM_BLOCK_TYPE = 32
NUM_ATOM_TYPE = 128
EMBED = 128

NC = 2
NS = 16
NW = NC * NS
L = 16

CHUNK = 128
NCHUNK = NU // CHUNK
BASE_CHUNKS = NCHUNK // NW
EXTRA = NCHUNK % NW


K = 5
SSTEP = NW * K
NSUPER = (NCHUNK + SSTEP - 1) // SSTEP


def _make_sc_gather():
    mesh = plsc.VectorSubcoreMesh(core_axis_name="c", subcore_axis_name="s")
    NSUPER2 = NSUPER + (NSUPER % 2)

    @functools.partial(
        pl.kernel,
        mesh=mesh,
        out_type=jax.ShapeDtypeStruct((NU, EMBED), jnp.float32),
        scratch_types=[
            pltpu.VMEM((K, CHUNK), jnp.int32),
            pltpu.VMEM((K, CHUNK), jnp.int32),
            pltpu.VMEM((K, CHUNK), jnp.int32),
            pltpu.VMEM((K, CHUNK), jnp.int32),
            pltpu.VMEM((K, CHUNK), jnp.int32),
            pltpu.VMEM((K, CHUNK), jnp.int32),
            pltpu.VMEM((K * CHUNK, EMBED), jnp.float32),
            pltpu.VMEM_SHARED(
                (NUM_BLOCK_TYPE * NUM_ATOM_TYPE, EMBED), jnp.float32
            ),
            pltpu.VMEM_SHARED((NB,), jnp.int32),
            pltpu.VMEM((2, EMBED), jnp.float32),
            pltpu.SemaphoreType.DMA,
            pltpu.SemaphoreType.DMA,
            pltpu.SemaphoreType.DMA,
            (pltpu.SemaphoreType.DMA,) * K,
            (pltpu.SemaphoreType.DMA,) * K,
        ],
    )
    def sc_gather(
        s_hbm, bid_hbm, a_hbm, blockt_hbm, atomt_hbm, out_hbm,
        bid0, bid1, a0, a1, t_v, f_v, rows_v, fused_sh, s_sh,
        blk_v,
        sin0, sin1, sem_t, sem_g, sem_o,
    ):
        bid_vs = (bid0, bid1)
        a_vs = (a0, a1)
        sem_in = (sin0, sin1)

        cid = lax.axis_index("c")
        sid = lax.axis_index("s")
        wid = sid * NC + cid

        def chunk_of(s, k):
            return s * SSTEP + wid * K + k

        def active(s, k):
            return jnp.logical_and(s >= 0, chunk_of(s, k) < NCHUNK)

        def row0_of(s, k):
            return chunk_of(s, k) * CHUNK

        def fire_in(s, b):
            for k in range(K):
                @pl.when(active(s, k))
                def _():
                    row0 = row0_of(s, k)
                    pltpu.async_copy(
                        bid_hbm.at[pl.ds(row0, CHUNK)], bid_vs[b].at[k], sem_in[b]
                    )
                    pltpu.async_copy(
                        a_hbm.at[pl.ds(row0, CHUNK)], a_vs[b].at[k], sem_in[b]
                    )

        def do_superstep(s, b):
            for k in range(K):
                @pl.when(active(s, k))
                def _():
                    row0 = row0_of(s, k)
                    pltpu.make_async_copy(
                        bid_hbm.at[pl.ds(row0, CHUNK)], bid_vs[b].at[k], sem_in[b]
                    ).wait()
                    pltpu.make_async_copy(
                        a_hbm.at[pl.ds(row0, CHUNK)], a_vs[b].at[k], sem_in[b]
                    ).wait()
                    pltpu.async_copy(s_sh.at[bid_vs[b].at[k]], t_v.at[k], sem_t)

            for k in range(K):
                @pl.when(active(s, k))
                def _():
                    pltpu.make_async_copy(
                        s_sh.at[bid_vs[b].at[k]], t_v.at[k], sem_t
                    ).wait()
                    for g in range(CHUNK // L):
                        t16 = t_v[k, pl.ds(g * L, L)]
                        a16 = a_vs[b][k, pl.ds(g * L, L)]
                        f_v[k, pl.ds(g * L, L)] = t16 * EMBED + a16

            fire_in(s + 2, b)

            for k in range(K):
                @pl.when(active(s - 1, k))
                def _():
                    pltpu.make_async_copy(
                        rows_v.at[pl.ds(k * CHUNK, CHUNK)],
                        out_hbm.at[pl.ds(row0_of(s - 1, k), CHUNK)],
                        sem_o[k],
                    ).wait()

                @pl.when(active(s, k))
                def _():
                    pltpu.async_copy(
                        fused_sh.at[f_v.at[k]],
                        rows_v.at[pl.ds(k * CHUNK, CHUNK)],
                        sem_g[k],
                    )

            for k in range(K):
                @pl.when(active(s, k))
                def _():
                    pltpu.make_async_copy(
                        fused_sh.at[f_v.at[k]],
                        rows_v.at[pl.ds(k * CHUNK, CHUNK)],
                        sem_g[k],
                    ).wait()
                    pltpu.async_copy(
                        rows_v.at[pl.ds(k * CHUNK, CHUNK)],
                        out_hbm.at[pl.ds(row0_of(s, k), CHUNK)],
                        sem_o[k],
                    )

        @pl.when(sid == 0)
        def _stage_s():
            pltpu.sync_copy(s_hbm, s_sh)

        AT0 = (K - 1) * CHUNK
        pltpu.sync_copy(atomt_hbm, rows_v.at[pl.ds(AT0, NUM_ATOM_TYPE)])
        pltpu.sync_copy(blockt_hbm.at[pl.ds(2 * sid, 2)], blk_v)
        blk_regs = [
            [blk_v[tl, pl.ds(g * L, L)] for g in range(EMBED // L)]
            for tl in range(2)
        ]

        @pl.loop(0, NUM_ATOM_TYPE // 8)
        def _build(ab):
            for da in range(8):
                a = ab * 8 + da
                for tl in range(2):
                    for g in range(EMBED // L):
                        rows_v[tl * EMBED + a, pl.ds(g * L, L)] = (
                            rows_v[AT0 + a, pl.ds(g * L, L)] + blk_regs[tl][g]
                        )

        pltpu.sync_copy(
            rows_v.at[pl.ds(0, 2 * EMBED)],
            fused_sh.at[pl.ds(sid * 2 * EMBED, 2 * EMBED)],
        )
        plsc.subcore_barrier()

        fire_in(0, 0)
        fire_in(1, 1)

        @pl.loop(0, NSUPER2, step=2)
        def _body(s0):
            for db in range(2):
                do_superstep(s0 + db, db)

        for k in range(K):
            @pl.when(active(NSUPER2 - 1, k))
            def _drain():
                pltpu.make_async_copy(
                    rows_v.at[pl.ds(k * CHUNK, CHUNK)],
                    out_hbm.at[pl.ds(row0_of(NSUPER2 - 1, k), CHUNK)],
                    sem_o[k],
                ).wait()

    return sc_gather


_sc_gather = _make_sc_gather()


@jax.jit
def kernel(S, A, block_id, block_table, atom_table):
    return _sc_gather(S, block_id, A, block_table, atom_table)


# speedup vs baseline: 1.3255x; 1.0012x over previous
import functools

import jax
import jax.numpy as jnp
from jax import lax
from jax.experimental import pallas as pl
from jax.experimental.pallas import tpu as pltpu
from jax.experimental.pallas import tpu_sc as plsc

NB = 50000
NU = 400000
NUM_BLOCK_TYPE = 32
NUM_ATOM_TYPE = 128
EMBED = 128

NC = 2
NS = 16
NW = NC * NS
L = 16

CHUNK = 128
NCHUNK = NU // CHUNK


K = 5
SSTEP = NW * K
NSUPER = (NCHUNK + SSTEP - 1) // SSTEP


def _make_sc_gather():
    mesh = plsc.VectorSubcoreMesh(core_axis_name="c", subcore_axis_name="s")
    NSUPER2 = NSUPER + (NSUPER % 2)

    @functools.partial(
        pl.kernel,
        mesh=mesh,
        out_type=jax.ShapeDtypeStruct((NU, EMBED), jnp.float32),
        scratch_types=[
            pltpu.VMEM((K, CHUNK), jnp.int32),
            pltpu.VMEM((K, CHUNK), jnp.int32),
            pltpu.VMEM((K, CHUNK), jnp.int32),
            pltpu.VMEM((K, CHUNK), jnp.int32),
            pltpu.VMEM((K, CHUNK), jnp.int32),
            pltpu.VMEM((K, CHUNK), jnp.int32),
            pltpu.VMEM((K * CHUNK, EMBED), jnp.float32),
            pltpu.VMEM_SHARED(
                (NUM_BLOCK_TYPE * NUM_ATOM_TYPE, EMBED), jnp.float32
            ),
            pltpu.VMEM_SHARED((NB,), jnp.int32),
            pltpu.VMEM((2, EMBED), jnp.float32),
            pltpu.SemaphoreType.DMA,
            pltpu.SemaphoreType.DMA,
            pltpu.SemaphoreType.DMA,
            (pltpu.SemaphoreType.DMA,) * K,
            (pltpu.SemaphoreType.DMA,) * K,
        ],
    )
    def sc_gather(
        s_hbm, bid_hbm, a_hbm, blockt_hbm, atomt_hbm, out_hbm,
        bid0, bid1, a0, a1, t_v, f_v, rows_v, fused_sh, s_sh,
        blk_v,
        sin0, sin1, sem_t, sem_g, sem_o,
    ):
        bid_vs = (bid0, bid1)
        a_vs = (a0, a1)
        sem_in = (sin0, sin1)

        cid = lax.axis_index("c")
        sid = lax.axis_index("s")
        wid = sid * NC + cid

        def chunk_of(s, k):
            return s * SSTEP + wid * K + k

        def active(s, k):
            return jnp.logical_and(s >= 0, chunk_of(s, k) < NCHUNK)

        def row0_of(s, k):
            return chunk_of(s, k) * CHUNK

        def fire_in(s, b):
            for k in range(K):
                @pl.when(active(s, k))
                def _():
                    row0 = row0_of(s, k)
                    pltpu.async_copy(
                        bid_hbm.at[pl.ds(row0, CHUNK)], bid_vs[b].at[k], sem_in[b]
                    )
                    pltpu.async_copy(
                        a_hbm.at[pl.ds(row0, CHUNK)], a_vs[b].at[k], sem_in[b]
                    )

        def do_superstep(s, b):
            for k in range(K):
                @pl.when(active(s, k))
                def _():
                    row0 = row0_of(s, k)
                    pltpu.make_async_copy(
                        bid_hbm.at[pl.ds(row0, CHUNK)], bid_vs[b].at[k], sem_in[b]
                    ).wait()
                    pltpu.make_async_copy(
                        a_hbm.at[pl.ds(row0, CHUNK)], a_vs[b].at[k], sem_in[b]
                    ).wait()
                    pltpu.async_copy(s_sh.at[bid_vs[b].at[k]], t_v.at[k], sem_t)

            for k in range(K):
                @pl.when(active(s, k))
                def _():
                    pltpu.make_async_copy(
                        s_sh.at[bid_vs[b].at[k]], t_v.at[k], sem_t
                    ).wait()
                    for g in range(CHUNK // L):
                        t16 = t_v[k, pl.ds(g * L, L)]
                        a16 = a_vs[b][k, pl.ds(g * L, L)]
                        f_v[k, pl.ds(g * L, L)] = t16 * EMBED + a16

            fire_in(s + 2, b)

            for k in range(K):
                @pl.when(active(s - 1, k))
                def _():
                    pltpu.make_async_copy(
                        rows_v.at[pl.ds(k * CHUNK, CHUNK)],
                        out_hbm.at[pl.ds(row0_of(s - 1, k), CHUNK)],
                        sem_o[k],
                    ).wait()

                @pl.when(active(s, k))
                def _():
                    pltpu.async_copy(
                        fused_sh.at[f_v.at[k]],
                        rows_v.at[pl.ds(k * CHUNK, CHUNK)],
                        sem_g[k],
                    )

            for k in range(K):
                @pl.when(active(s, k))
                def _():
                    pltpu.make_async_copy(
                        fused_sh.at[f_v.at[k]],
                        rows_v.at[pl.ds(k * CHUNK, CHUNK)],
                        sem_g[k],
                    ).wait()
                    pltpu.async_copy(
                        rows_v.at[pl.ds(k * CHUNK, CHUNK)],
                        out_hbm.at[pl.ds(row0_of(s, k), CHUNK)],
                        sem_o[k],
                    )

        @pl.when(sid == 0)
        def _stage_s():
            pltpu.sync_copy(s_hbm, s_sh)

        AT0 = (K - 1) * CHUNK
        pltpu.sync_copy(atomt_hbm, rows_v.at[pl.ds(AT0, NUM_ATOM_TYPE)])
        pltpu.sync_copy(blockt_hbm.at[pl.ds(2 * sid, 2)], blk_v)
        blk_regs = [
            [blk_v[tl, pl.ds(g * L, L)] for g in range(EMBED // L)]
            for tl in range(2)
        ]

        @pl.loop(0, NUM_ATOM_TYPE // 8)
        def _build(ab):
            for da in range(8):
                a = ab * 8 + da
                for tl in range(2):
                    for g in range(EMBED // L):
                        rows_v[tl * EMBED + a, pl.ds(g * L, L)] = (
                            rows_v[AT0 + a, pl.ds(g * L, L)] + blk_regs[tl][g]
                        )

        pltpu.sync_copy(
            rows_v.at[pl.ds(0, 2 * EMBED)],
            fused_sh.at[pl.ds(sid * 2 * EMBED, 2 * EMBED)],
        )
        plsc.subcore_barrier()

        fire_in(0, 0)
        fire_in(1, 1)

        @pl.loop(0, NSUPER2, step=2)
        def _body(s0):
            for db in range(2):
                do_superstep(s0 + db, db)

        for k in range(K):
            @pl.when(active(NSUPER2 - 1, k))
            def _drain():
                pltpu.make_async_copy(
                    rows_v.at[pl.ds(k * CHUNK, CHUNK)],
                    out_hbm.at[pl.ds(row0_of(NSUPER2 - 1, k), CHUNK)],
                    sem_o[k],
                ).wait()

    return sc_gather


_sc_gather = _make_sc_gather()


@jax.jit
def kernel(S, A, block_id, block_table, atom_table):
    return _sc_gather(S, block_id, A, block_table, atom_table)
